# baseline (speedup 1.0000x reference)
import functools

import jax
import jax.numpy as jnp
from jax import lax
from jax.experimental import pallas as pl
from jax.experimental.pallas import tpu as pltpu
from jax.experimental.pallas import tpu_sc as plsc

NUM_CORES = 2
NUM_SUBCORES = 16
LANES = 16
NUM_WORKERS = NUM_CORES * NUM_SUBCORES

BATCH = 16384
EMBED = 16
VOCAB = 1000000

BPW = BATCH // NUM_WORKERS
CHUNK = 128
NCHUNKS = BPW // CHUNK

TC_COLS = 262144
TC_GRID = -(-VOCAB // TC_COLS)
S_ROWS = TC_GRID * (TC_COLS // 128)


def _tc_body(w_sref, t_ref, o_ref):
    acc = jnp.zeros((TC_COLS // 128, 128), jnp.float32)
    for d in range(EMBED):
        acc = acc + t_ref[d].reshape(TC_COLS // 128, 128) * w_sref[0, d]
    o_ref[...] = acc


_tc_matvec = pl.pallas_call(
    _tc_body,
    grid=(TC_GRID,),
    in_specs=[
        pl.BlockSpec(memory_space=pltpu.SMEM),
        pl.BlockSpec((EMBED, TC_COLS), lambda g: (0, g)),
    ],
    out_specs=pl.BlockSpec((TC_COLS // 128, 128), lambda g: (g, 0)),
    out_shape=jax.ShapeDtypeStruct((S_ROWS, 128), jnp.float32),
)


def _sc_body(x_hbm, s_hbm, params_hbm, out_hbm, idx_v, sup_v, sub_v,
             rows_v, out_v, par_v, sem):
    wid = lax.axis_index("s") * NUM_CORES + lax.axis_index("c")
    base = wid * BPW

    pltpu.sync_copy(params_hbm, par_v)
    pltpu.sync_copy(x_hbm.at[pl.ds(base, BPW)], idx_v)

    for k in range(BPW // LANES):
        sl = pl.ds(k * LANES, LANES)
        v = idx_v[sl]
        sup_v[sl] = lax.shift_right_logical(v, 7)
        sub_v[sl] = v & 127

    copies = [
        pltpu.async_copy(
            s_hbm.at[sup_v.at[pl.ds(j * CHUNK, CHUNK)]],
            rows_v.at[pl.ds(j * CHUNK, CHUNK)],
            sem.at[j],
        )
        for j in range(NCHUNKS)
    ]

    lane = lax.iota(jnp.int32, LANES)
    bias = par_v[pl.ds(0, LANES)][0]

    def block(t, carry):
        rvec = t * LANES + lane
        subvec = sub_v[pl.ds(t * LANES, LANES)]
        out_v[pl.ds(t * LANES, LANES)] = (
            plsc.load_gather(rows_v, [rvec, subvec]) + bias
        )
        return carry

    blocks_per_chunk = CHUNK // LANES
    for j in range(NCHUNKS):
        copies[j].wait()
        lax.fori_loop(j * blocks_per_chunk, (j + 1) * blocks_per_chunk,
                      block, 0)

    pltpu.sync_copy(out_v, out_hbm.at[pl.ds(base, BPW)])


@functools.partial(
    pl.kernel,
    out_type=jax.ShapeDtypeStruct((BATCH,), jnp.float32),
    mesh=plsc.VectorSubcoreMesh(core_axis_name="c", subcore_axis_name="s"),
    scratch_types=[
        pltpu.VMEM((BPW,), jnp.int32),
        pltpu.VMEM((BPW,), jnp.int32),
        pltpu.VMEM((BPW,), jnp.int32),
        pltpu.VMEM((BPW, 128), jnp.float32),
        pltpu.VMEM((BPW,), jnp.float32),
        pltpu.VMEM((LANES,), jnp.float32),
        pltpu.SemaphoreType.DMA((NCHUNKS,)),
    ],
    compiler_params=pltpu.CompilerParams(needs_layout_passes=False),
)
def _sc_gather(x_hbm, s_hbm, params_hbm, out_hbm, idx_v, sup_v, sub_v,
               rows_v, out_v, par_v, sem):
    _sc_body(x_hbm, s_hbm, params_hbm, out_hbm, idx_v, sup_v, sub_v,
             rows_v, out_v, par_v, sem)


def kernel(x, table, fc_w, fc_b):
    xi = x.astype(jnp.int32)
    tt = table.astype(jnp.float32).T
    w = fc_w.reshape(1, EMBED).astype(jnp.float32)
    s2d = _tc_matvec(w, tt)
    pbias = jnp.broadcast_to(fc_b.astype(jnp.float32).reshape(1), (LANES,))
    out = _sc_gather(xi, s2d, pbias)
    return out.reshape(x.shape[0], 1)


# speedup vs baseline: 1.0449x; 1.0449x over previous
import functools

import jax
import jax.numpy as jnp
from jax import lax
from jax.experimental import pallas as pl
from jax.experimental.pallas import tpu as pltpu
from jax.experimental.pallas import tpu_sc as plsc

NUM_CORES = 2
NUM_SUBCORES = 16
LANES = 16
NUM_WORKERS = NUM_CORES * NUM_SUBCORES

BATCH = 16384
EMBED = 16
VOCAB = 1000000

BPW = BATCH // NUM_WORKERS
CHUNK = 128
NCHUNKS = BPW // CHUNK

TC_COLS = 131072
TC_GRID = -(-VOCAB // TC_COLS)
S_ROWS = TC_GRID * (TC_COLS // 128)


def _tc_body(w_sref, t_ref, o_ref):
    acc = jnp.zeros((TC_COLS // 128, 128), jnp.float32)
    for d in range(EMBED):
        acc = acc + t_ref[d].reshape(TC_COLS // 128, 128) * w_sref[0, d]
    o_ref[...] = acc


_tc_matvec = pl.pallas_call(
    _tc_body,
    grid=(TC_GRID,),
    in_specs=[
        pl.BlockSpec(memory_space=pltpu.SMEM),
        pl.BlockSpec((EMBED, TC_COLS), lambda g: (0, g)),
    ],
    out_specs=pl.BlockSpec((TC_COLS // 128, 128), lambda g: (g, 0)),
    out_shape=jax.ShapeDtypeStruct((S_ROWS, 128), jnp.float32),
)


def _sc_body(x_hbm, s_hbm, params_hbm, out_hbm, idx_v, sup_v, sub_v,
             rows_v, out_v, par_v, sem):
    wid = lax.axis_index("s") * NUM_CORES + lax.axis_index("c")
    base = wid * BPW

    pltpu.sync_copy(params_hbm, par_v)
    pltpu.sync_copy(x_hbm.at[pl.ds(base, BPW)], idx_v)

    for k in range(BPW // LANES):
        sl = pl.ds(k * LANES, LANES)
        v = idx_v[sl]
        sup_v[sl] = lax.shift_right_logical(v, 7)
        sub_v[sl] = v & 127

    copies = [
        pltpu.async_copy(
            s_hbm.at[sup_v.at[pl.ds(j * CHUNK, CHUNK)]],
            rows_v.at[pl.ds(j * CHUNK, CHUNK)],
            sem.at[j],
        )
        for j in range(NCHUNKS)
    ]

    lane = lax.iota(jnp.int32, LANES)
    bias = par_v[pl.ds(0, LANES)][0]

    def block(t, carry):
        rvec = t * LANES + lane
        subvec = sub_v[pl.ds(t * LANES, LANES)]
        out_v[pl.ds(t * LANES, LANES)] = (
            plsc.load_gather(rows_v, [rvec, subvec]) + bias
        )
        return carry

    blocks_per_chunk = CHUNK // LANES
    for j in range(NCHUNKS):
        copies[j].wait()
        lax.fori_loop(j * blocks_per_chunk, (j + 1) * blocks_per_chunk,
                      block, 0)

    pltpu.sync_copy(out_v, out_hbm.at[pl.ds(base, BPW)])


@functools.partial(
    pl.kernel,
    out_type=jax.ShapeDtypeStruct((BATCH,), jnp.float32),
    mesh=plsc.VectorSubcoreMesh(core_axis_name="c", subcore_axis_name="s"),
    scratch_types=[
        pltpu.VMEM((BPW,), jnp.int32),
        pltpu.VMEM((BPW,), jnp.int32),
        pltpu.VMEM((BPW,), jnp.int32),
        pltpu.VMEM((BPW, 128), jnp.float32),
        pltpu.VMEM((BPW,), jnp.float32),
        pltpu.VMEM((LANES,), jnp.float32),
        pltpu.SemaphoreType.DMA((NCHUNKS,)),
    ],
    compiler_params=pltpu.CompilerParams(needs_layout_passes=False),
)
def _sc_gather(x_hbm, s_hbm, params_hbm, out_hbm, idx_v, sup_v, sub_v,
               rows_v, out_v, par_v, sem):
    _sc_body(x_hbm, s_hbm, params_hbm, out_hbm, idx_v, sup_v, sub_v,
             rows_v, out_v, par_v, sem)


def kernel(x, table, fc_w, fc_b):
    xi = x.astype(jnp.int32)
    tt = table.astype(jnp.float32).T
    w = fc_w.reshape(1, EMBED).astype(jnp.float32)
    s2d = _tc_matvec(w, tt)
    pbias = jnp.broadcast_to(fc_b.astype(jnp.float32).reshape(1), (LANES,))
    out = _sc_gather(xi, s2d, pbias)
    return out.reshape(x.shape[0], 1)
